# Initial kernel scaffold; baseline (speedup 1.0000x reference)
import jax
import jax.numpy as jnp
from jax.experimental import pallas as pl


def kernel(x, edge_index, vit_output, W1, b1, W2, b2, W3, b3, W4, b4, g1, be1, g2, be2, Wp, bp, Wl, bl):
    raise NotImplementedError("write your pallas kernel here")


# speedup vs baseline: 18.2699x; 18.2699x over previous
import jax
import jax.numpy as jnp
from jax import lax
from jax.experimental import pallas as pl
from jax.experimental.pallas import tpu as pltpu
from jax.experimental.pallas import tpu_sc as plsc

N = 50176
E = 802816
EB = E // 128
NSC = 2
NTILE = 16
RPT = N // NTILE
NB = 4
BN_EPS = 1e-5
RB = 512
NRB = N // RB

_MESH = plsc.VectorSubcoreMesh(core_axis_name="c", subcore_axis_name="s",
                               num_cores=NSC, num_subcores=NTILE)


def _make_deg():
    def body(dst_hbm, out_hbm, acc, dstv, ones, zbuf, dstage):
        cid = lax.axis_index("c")
        sid = lax.axis_index("s")
        for i in range(8):
            ones[pl.ds(16 * i, 16)] = jnp.full((16,), 1.0, jnp.float32)
        for i in range(14):
            zbuf[pl.ds(16 * i, 16)] = jnp.zeros((16,), jnp.float32)
        r0 = sid * RPT
        for k in range(RPT // 224):
            pltpu.sync_copy(zbuf, acc.at[pl.ds(r0 + k * 224, 224)])
        plsc.subcore_barrier()
        nch_t = 24 + jnp.where(sid < 8, 1, 0)
        tb = cid * (EB // NSC) + (sid * 24 + jnp.minimum(sid, 8)) * 8

        def chunk(ci, carry):
            cb = tb + ci * 8
            pltpu.sync_copy(dst_hbm.at[pl.ds(cb, 8)], dstv)

            def bat(j, c2):
                pltpu.sync_copy(ones, acc.at[dstv.at[j]], add=True)
                return c2
            return lax.fori_loop(0, 8, bat, carry)
        lax.fori_loop(0, nch_t, chunk, 0)
        plsc.subcore_barrier()
        pltpu.sync_copy(acc.at[pl.ds(r0, RPT)], dstage)
        pltpu.sync_copy(dstage, out_hbm.at[pl.ds(cid * N + r0, RPT)])

    return pl.kernel(
        body,
        out_type=jax.ShapeDtypeStruct((NSC * N,), jnp.float32),
        mesh=_MESH,
        scratch_types=[
            pltpu.VMEM_SHARED((N,), jnp.float32),
            pltpu.VMEM((8, 128), jnp.int32),
            pltpu.VMEM((128,), jnp.float32),
            pltpu.VMEM((224,), jnp.float32),
            pltpu.VMEM((RPT,), jnp.float32),
        ],
        compiler_params=pltpu.CompilerParams(use_tc_tiling_on_sc=False),
    )


def _make_spmm(sp, w, edge_split):
    shift = sp.bit_length() - 1
    blocks_per_sc = 1 if edge_split else (128 // w) // NSC
    nch = 8 if edge_split else 28
    sro = 224 if edge_split else 112

    def body(gf_hbm, g3_hbm, src_hbm, dst_hbm, out_hbm, acc, srcv, dstv,
             idx0, idx1, idx2, idx3, rows0, rows1, rows2, rows3, stage,
             gsem0, gsem1, gsem2, gsem3, ssem0, ssem1, ssem2, ssem3):
        cid = lax.axis_index("c")
        sid = lax.axis_index("s")
        r0 = sid * RPT
        rows = (rows0, rows1, rows2, rows3)
        idxb = (idx0, idx1, idx2, idx3)
        gsem = (gsem0, gsem1, gsem2, gsem3)
        ssem = (ssem0, ssem1, ssem2, ssem3)
        for t in range(blocks_per_sc):
            if edge_split:
                part = cid
                gpart = 0
                nch_t = 24 + jnp.where(sid < 8, 1, 0)
                tb = cid * (EB // NSC) + (sid * 24 + jnp.minimum(sid, 8)) * 8
            else:
                part = cid * blocks_per_sc + t
                gpart = part
                nch_t = (EB // NTILE) // nch
                tb = sid * (EB // NTILE)
            for k in range(RPT // sro):
                pltpu.sync_copy(g3_hbm.at[pl.ds(r0 + k * sro, sro), gpart, :],
                                stage)
                pltpu.sync_copy(stage, acc.at[pl.ds(r0 + k * sro, sro)])
            plsc.subcore_barrier()

            def issue_gather(j, b):
                for i in range(8):
                    v = srcv[j, pl.ds(16 * i, 16)] << shift
                    if not edge_split:
                        v = v + gpart
                    idxb[b][pl.ds(16 * i, 16)] = v
                return pltpu.async_copy(gf_hbm.at[idxb[b]], rows[b], gsem[b])

            def chunk(ci, carry):
                cb = tb + ci * nch
                c1 = pltpu.async_copy(src_hbm.at[pl.ds(cb, nch)], srcv,
                                      gsem0)
                c2 = pltpu.async_copy(dst_hbm.at[pl.ds(cb, nch)], dstv,
                                      gsem1)
                c1.wait()
                c2.wait()
                gh = [None] * NB
                sh = [None] * NB
                for k in range(min(NB - 1, nch)):
                    gh[k] = issue_gather(k, k)
                for j in range(nch):
                    b = j % NB
                    gh[b].wait()
                    sh[b] = pltpu.async_copy(rows[b], acc.at[dstv.at[j]],
                                             ssem[b], add=True)
                    if j >= 1:
                        sh[(j - 1) % NB].wait()
                    if j + NB - 1 < nch:
                        b2 = (j + NB - 1) % NB
                        gh[b2] = issue_gather(j + NB - 1, b2)
                sh[(nch - 1) % NB].wait()
                return carry
            lax.fori_loop(0, nch_t, chunk, 0)
            plsc.subcore_barrier()
            for k in range(RPT // sro):
                pltpu.sync_copy(acc.at[pl.ds(r0 + k * sro, sro)], stage)
                pltpu.sync_copy(
                    stage,
                    out_hbm.at[pl.ds(r0 + k * sro, sro), pl.ds(part * w, w)])

    return pl.kernel(
        body,
        out_type=jax.ShapeDtypeStruct((N, 128), jnp.float32),
        mesh=_MESH,
        scratch_types=[
            pltpu.VMEM_SHARED((N, w), jnp.float32),
            pltpu.VMEM((nch, 128), jnp.int32),
            pltpu.VMEM((nch, 128), jnp.int32),
            pltpu.VMEM((128,), jnp.int32),
            pltpu.VMEM((128,), jnp.int32),
            pltpu.VMEM((128,), jnp.int32),
            pltpu.VMEM((128,), jnp.int32),
            pltpu.VMEM((128, w), jnp.float32),
            pltpu.VMEM((128, w), jnp.float32),
            pltpu.VMEM((128, w), jnp.float32),
            pltpu.VMEM((128, w), jnp.float32),
            pltpu.VMEM((sro, w), jnp.float32),
            pltpu.SemaphoreType.DMA,
            pltpu.SemaphoreType.DMA,
            pltpu.SemaphoreType.DMA,
            pltpu.SemaphoreType.DMA,
            pltpu.SemaphoreType.DMA,
            pltpu.SemaphoreType.DMA,
            pltpu.SemaphoreType.DMA,
            pltpu.SemaphoreType.DMA,
        ],
        compiler_params=pltpu.CompilerParams(use_tc_tiling_on_sc=False),
    )


_deg = _make_deg()
_spmm_l1 = _make_spmm(8, 16, True)
_spmm_l2 = _make_spmm(4, 32, True)
_spmm_l3 = _make_spmm(4, 32, False)
_spmm_l4 = _make_spmm(4, 32, False)


def _rb(i):
    return (i, 0)


def _full(i):
    return (0, 0)


def _zeros(nlanes):
    return jnp.zeros((RB, nlanes), jnp.float32)


def _tc1_body(d_ref, x_ref, w1_ref, out_ref):
    deg = d_ref[0] + d_ref[1] + 1.0
    dinv = lax.rsqrt(deg)
    g1 = jnp.dot(x_ref[...], w1_ref[...]) * dinv
    out_ref[...] = jnp.concatenate([g1, dinv, _zeros(111)], axis=1)


def _tc1(dstk, x, w1):
    return pl.pallas_call(
        _tc1_body,
        grid=(NRB,),
        in_specs=[
            pl.BlockSpec((NSC, RB, 1), lambda i: (0, i, 0)),
            pl.BlockSpec((RB, 64), _rb),
            pl.BlockSpec((64, 16), _full),
        ],
        out_specs=pl.BlockSpec((RB, 128), _rb),
        out_shape=jax.ShapeDtypeStruct((N, 128), jnp.float32),
    )(dstk, x, w1)


def _tc2_body(s_ref, g_ref, b_ref, w_ref, out_ref):
    dinv = g_ref[:, 16:17]
    s = s_ref[:, :16] + s_ref[:, 16:32] - g_ref[:, :16]
    h = jnp.maximum(s * dinv + b_ref[...], 0.0)
    g2 = jnp.dot(h, w_ref[...]) * dinv
    out_ref[...] = jnp.concatenate([g2, dinv, _zeros(95)], axis=1)


def _tc2(s1, g1pk, b1, w2):
    return pl.pallas_call(
        _tc2_body,
        grid=(NRB,),
        in_specs=[
            pl.BlockSpec((RB, 128), _rb),
            pl.BlockSpec((RB, 128), _rb),
            pl.BlockSpec((1, 16), _full),
            pl.BlockSpec((16, 32), _full),
        ],
        out_specs=pl.BlockSpec((RB, 128), _rb),
        out_shape=jax.ShapeDtypeStruct((N, 128), jnp.float32),
    )(s1, g1pk, b1, w2)


def _tc3a_body(s_ref, g_ref, b_ref, y_ref, p_ref):
    dinv = g_ref[:, 32:33]
    y = (s_ref[:, :32] + s_ref[:, 32:64] - g_ref[:, :32]) * dinv + b_ref[...]
    y_ref[...] = jnp.concatenate([y, dinv, _zeros(95)], axis=1)
    p_ref[0, 0:1, :] = jnp.sum(y, axis=0, keepdims=True)
    p_ref[0, 1:2, :] = jnp.sum(y * y, axis=0, keepdims=True)


def _tc3a(s2, g2pk, b2):
    return pl.pallas_call(
        _tc3a_body,
        grid=(NRB,),
        in_specs=[
            pl.BlockSpec((RB, 128), _rb),
            pl.BlockSpec((RB, 128), _rb),
            pl.BlockSpec((1, 32), _full),
        ],
        out_specs=[
            pl.BlockSpec((RB, 128), _rb),
            pl.BlockSpec((1, 2, 32), lambda i: (i, 0, 0)),
        ],
        out_shape=[
            jax.ShapeDtypeStruct((N, 128), jnp.float32),
            jax.ShapeDtypeStruct((NRB, 2, 32), jnp.float32),
        ],
    )(s2, g2pk, b2)


def _tc3b_body(y_ref, p_ref, gm_ref, bt_ref, w_ref, out_ref):
    dinv = y_ref[:, 32:33]
    m = jnp.sum(p_ref[:, 0, :], axis=0, keepdims=True) / N
    v = jnp.sum(p_ref[:, 1, :], axis=0, keepdims=True) / N - m * m
    scale = gm_ref[...] * lax.rsqrt(v + BN_EPS)
    h = jnp.maximum((y_ref[:, :32] - m) * scale + bt_ref[...], 0.0)
    g3 = jnp.dot(h, w_ref[...]) * dinv
    out_ref[...] = jnp.concatenate([g3, dinv, _zeros(63)], axis=1)


def _tc3b(y2, p2, gm, bt, w3):
    return pl.pallas_call(
        _tc3b_body,
        grid=(NRB,),
        in_specs=[
            pl.BlockSpec((RB, 128), _rb),
            pl.BlockSpec((NRB, 2, 32), lambda i: (0, 0, 0)),
            pl.BlockSpec((1, 32), _full),
            pl.BlockSpec((1, 32), _full),
            pl.BlockSpec((32, 64), _full),
        ],
        out_specs=pl.BlockSpec((RB, 128), _rb),
        out_shape=jax.ShapeDtypeStruct((N, 128), jnp.float32),
    )(y2, p2, gm, bt, w3)


def _tc4_body(s_ref, g_ref, b_ref, w_ref, out_ref):
    dinv = g_ref[:, 64:65]
    h = jnp.maximum(s_ref[:, :64] * dinv + b_ref[...], 0.0)
    out_ref[...] = jnp.dot(h, w_ref[...]) * dinv


def _tc4(s3, g3pk, b3, w4):
    return pl.pallas_call(
        _tc4_body,
        grid=(NRB,),
        in_specs=[
            pl.BlockSpec((RB, 128), _rb),
            pl.BlockSpec((RB, 128), _rb),
            pl.BlockSpec((1, 64), _full),
            pl.BlockSpec((64, 128), _full),
        ],
        out_specs=pl.BlockSpec((RB, 128), _rb),
        out_shape=jax.ShapeDtypeStruct((N, 128), jnp.float32),
    )(s3, g3pk, b3, w4)


def _tc5a_body(s_ref, g_ref, b_ref, y_ref, p_ref):
    dinv = g_ref[:, 64:65]
    y = s_ref[...] * dinv + b_ref[...]
    y_ref[...] = y
    p_ref[0, 0:1, :] = jnp.sum(y, axis=0, keepdims=True)
    p_ref[0, 1:2, :] = jnp.sum(y * y, axis=0, keepdims=True)


def _tc5a(s4, g3pk, b4):
    return pl.pallas_call(
        _tc5a_body,
        grid=(NRB,),
        in_specs=[
            pl.BlockSpec((RB, 128), _rb),
            pl.BlockSpec((RB, 128), _rb),
            pl.BlockSpec((1, 128), _full),
        ],
        out_specs=[
            pl.BlockSpec((RB, 128), _rb),
            pl.BlockSpec((1, 2, 128), lambda i: (i, 0, 0)),
        ],
        out_shape=[
            jax.ShapeDtypeStruct((N, 128), jnp.float32),
            jax.ShapeDtypeStruct((NRB, 2, 128), jnp.float32),
        ],
    )(s4, g3pk, b4)


def _tc5b_body(y_ref, p_ref, gm_ref, bt_ref, wp_ref, bp_ref, out_ref):
    i = pl.program_id(0)
    m = jnp.sum(p_ref[:, 0, :], axis=0, keepdims=True) / N
    v = jnp.sum(p_ref[:, 1, :], axis=0, keepdims=True) / N - m * m
    scale = gm_ref[...] * lax.rsqrt(v + BN_EPS)
    h = jnp.maximum((y_ref[...] - m) * scale + bt_ref[...], 0.0)

    @pl.when(i == 0)
    def _():
        out_ref[...] = jnp.broadcast_to(bp_ref[...], (196, 128))

    out_ref[...] += jnp.dot(wp_ref[...], h)


def _tc5b(y4, p4, gm, bt, wp, bp):
    return pl.pallas_call(
        _tc5b_body,
        grid=(NRB,),
        in_specs=[
            pl.BlockSpec((RB, 128), _rb),
            pl.BlockSpec((NRB, 2, 128), lambda i: (0, 0, 0)),
            pl.BlockSpec((1, 128), _full),
            pl.BlockSpec((1, 128), _full),
            pl.BlockSpec((196, RB), lambda i: (0, i)),
            pl.BlockSpec((196, 1), _full),
        ],
        out_specs=pl.BlockSpec((196, 128), _full),
        out_shape=jax.ShapeDtypeStruct((196, 128), jnp.float32),
    )(y4, p4, gm, bt, wp, bp)


def _tc6_body(vit_ref, p_ref, wl_ref, bl_ref, out_ref):
    wa = wl_ref[:, :768]
    wb = wl_ref[:, 768:]
    dn = (((1,), (1,)), ((), ()))
    out_ref[...] = (lax.dot_general(vit_ref[...], wa, dn)
                    + lax.dot_general(p_ref[...], wb, dn)
                    + bl_ref[...])


def _tc6(vit, p, wl, bl):
    return pl.pallas_call(
        _tc6_body,
        grid=(1,),
        in_specs=[
            pl.BlockSpec((196, 768), _full),
            pl.BlockSpec((196, 128), _full),
            pl.BlockSpec((768, 896), _full),
            pl.BlockSpec((1, 768), _full),
        ],
        out_specs=pl.BlockSpec((196, 768), _full),
        out_shape=jax.ShapeDtypeStruct((196, 768), jnp.float32),
    )(vit, p, wl, bl)


def kernel(x, edge_index, vit_output, W1, b1, W2, b2, W3, b3, W4, b4,
           g1, be1, g2, be2, Wp, bp, Wl, bl):
    src2d = edge_index[0].reshape(EB, 128)
    dst2d = edge_index[1].reshape(EB, 128)

    degp = _deg(dst2d)
    dstk = degp.reshape(NSC, N, 1)
    g1pk = _tc1(dstk, x, W1)

    s1 = _spmm_l1(g1pk.reshape(8 * N, 16), g1pk.reshape(N, 8, 16),
                  src2d, dst2d)
    g2pk = _tc2(s1, g1pk, b1.reshape(1, 16), W2)

    s2 = _spmm_l2(g2pk.reshape(4 * N, 32), g2pk.reshape(N, 4, 32),
                  src2d, dst2d)
    y2pk, p2 = _tc3a(s2, g2pk, b2.reshape(1, 32))
    g3pk = _tc3b(y2pk, p2, g1.reshape(1, 32), be1.reshape(1, 32), W3)

    s3 = _spmm_l3(g3pk.reshape(4 * N, 32), g3pk.reshape(N, 4, 32),
                  src2d, dst2d)
    g4pk = _tc4(s3, g3pk, b3.reshape(1, 64), W4)

    s4 = _spmm_l4(g4pk.reshape(4 * N, 32), g4pk.reshape(N, 4, 32),
                  src2d, dst2d)
    y4, p4 = _tc5a(s4, g3pk, b4.reshape(1, 128))
    pmat = _tc5b(y4, p4, g2.reshape(1, 128), be2.reshape(1, 128),
                 Wp, bp.reshape(196, 1))

    z = _tc6(vit_output.reshape(196, 768), pmat, Wl, bl.reshape(1, 768))
    return z.reshape(1, 196, 768)


# speedup vs baseline: 24.2264x; 1.3260x over previous
import jax
import jax.numpy as jnp
from jax import lax
from jax.experimental import pallas as pl
from jax.experimental.pallas import tpu as pltpu
from jax.experimental.pallas import tpu_sc as plsc

N = 50176
E = 802816
EB = E // 128
NSC = 2
NTILE = 16
RPT = N // NTILE
NB = 4
BN_EPS = 1e-5
RB = 512
NRB = N // RB

_MESH = plsc.VectorSubcoreMesh(core_axis_name="c", subcore_axis_name="s",
                               num_cores=NSC, num_subcores=NTILE)


def _make_deg():
    def body(dst_hbm, out_hbm, acc, dstv, ones, zbuf, dstage):
        cid = lax.axis_index("c")
        sid = lax.axis_index("s")
        for i in range(8):
            ones[pl.ds(16 * i, 16)] = jnp.full((16,), 1.0, jnp.float32)
        for i in range(14):
            zbuf[pl.ds(16 * i, 16)] = jnp.zeros((16,), jnp.float32)
        r0 = sid * RPT
        for k in range(RPT // 224):
            pltpu.sync_copy(zbuf, acc.at[pl.ds(r0 + k * 224, 224)])
        plsc.subcore_barrier()
        nch_t = 24 + jnp.where(sid < 8, 1, 0)
        tb = cid * (EB // NSC) + (sid * 24 + jnp.minimum(sid, 8)) * 8

        def chunk(ci, carry):
            cb = tb + ci * 8
            pltpu.sync_copy(dst_hbm.at[pl.ds(cb, 8)], dstv)

            def bat(j, c2):
                pltpu.sync_copy(ones, acc.at[dstv.at[j]], add=True)
                return c2
            return lax.fori_loop(0, 8, bat, carry)
        lax.fori_loop(0, nch_t, chunk, 0)
        plsc.subcore_barrier()
        pltpu.sync_copy(acc.at[pl.ds(r0, RPT)], dstage)
        pltpu.sync_copy(dstage, out_hbm.at[pl.ds(cid * N + r0, RPT)])

    return pl.kernel(
        body,
        out_type=jax.ShapeDtypeStruct((NSC * N,), jnp.float32),
        mesh=_MESH,
        scratch_types=[
            pltpu.VMEM_SHARED((N,), jnp.float32),
            pltpu.VMEM((8, 128), jnp.int32),
            pltpu.VMEM((128,), jnp.float32),
            pltpu.VMEM((224,), jnp.float32),
            pltpu.VMEM((RPT,), jnp.float32),
        ],
        compiler_params=pltpu.CompilerParams(use_tc_tiling_on_sc=False),
    )


def _make_spmm(sp, w, edge_split):
    shift = sp.bit_length() - 1
    blocks_per_sc = 1 if edge_split else (128 // w) // NSC
    nch = 8 if edge_split else 28
    sro = 224 if edge_split else 112

    def body(gf_hbm, src_hbm, dst_hbm, out_hbm, acc, srcv, dstv,
             idx0, idx1, idx2, idx3, rows0, rows1, rows2, rows3, stage,
             gsem0, gsem1, gsem2, gsem3, ssem0, ssem1, ssem2, ssem3):
        cid = lax.axis_index("c")
        sid = lax.axis_index("s")
        r0 = sid * RPT
        rows = (rows0, rows1, rows2, rows3)
        idxb = (idx0, idx1, idx2, idx3)
        gsem = (gsem0, gsem1, gsem2, gsem3)
        ssem = (ssem0, ssem1, ssem2, ssem3)
        for t in range(blocks_per_sc):
            if edge_split:
                part = cid
                gpart = 0
                nch_t = 24 + jnp.where(sid < 8, 1, 0)
                tb = cid * (EB // NSC) + (sid * 24 + jnp.minimum(sid, 8)) * 8
            else:
                part = cid * blocks_per_sc + t
                gpart = part
                nch_t = (EB // NTILE) // nch
                tb = sid * (EB // NTILE)
            lanes = lax.iota(jnp.int32, 16)

            def initk(k, carry):
                base = r0 + k * 112
                for i in range(7):
                    idx0[pl.ds(16 * i, 16)] = (
                        ((base + 16 * i + lanes) << shift) + gpart)
                pltpu.async_copy(gf_hbm.at[idx0.at[pl.ds(0, 112)]],
                                 stage.at[pl.ds(0, 112)], gsem0).wait()
                pltpu.sync_copy(stage.at[pl.ds(0, 112)],
                                acc.at[pl.ds(base, 112)])
                return carry
            lax.fori_loop(0, RPT // 112, initk, 0)
            plsc.subcore_barrier()

            def issue_gather(j, b):
                for i in range(8):
                    v = srcv[j, pl.ds(16 * i, 16)] << shift
                    if not edge_split:
                        v = v + gpart
                    idxb[b][pl.ds(16 * i, 16)] = v
                return pltpu.async_copy(gf_hbm.at[idxb[b]], rows[b], gsem[b])

            def chunk(ci, carry):
                cb = tb + ci * nch
                c1 = pltpu.async_copy(src_hbm.at[pl.ds(cb, nch)], srcv,
                                      gsem0)
                c2 = pltpu.async_copy(dst_hbm.at[pl.ds(cb, nch)], dstv,
                                      gsem1)
                c1.wait()
                c2.wait()
                gh = [None] * NB
                sh = [None] * NB
                for k in range(min(NB - 1, nch)):
                    gh[k] = issue_gather(k, k)
                for j in range(nch):
                    b = j % NB
                    gh[b].wait()
                    sh[b] = pltpu.async_copy(rows[b], acc.at[dstv.at[j]],
                                             ssem[b], add=True)
                    if j >= 1:
                        sh[(j - 1) % NB].wait()
                    if j + NB - 1 < nch:
                        b2 = (j + NB - 1) % NB
                        gh[b2] = issue_gather(j + NB - 1, b2)
                sh[(nch - 1) % NB].wait()
                return carry
            lax.fori_loop(0, nch_t, chunk, 0)
            plsc.subcore_barrier()

            def outk(k, carry):
                pltpu.sync_copy(acc.at[pl.ds(r0 + k * sro, sro)], stage)
                pltpu.sync_copy(
                    stage,
                    out_hbm.at[pl.ds(r0 + k * sro, sro), pl.ds(part * w, w)])
                return carry
            lax.fori_loop(0, RPT // sro, outk, 0)

    return pl.kernel(
        body,
        out_type=jax.ShapeDtypeStruct((N, 128), jnp.float32),
        mesh=_MESH,
        scratch_types=[
            pltpu.VMEM_SHARED((N, w), jnp.float32),
            pltpu.VMEM((nch, 128), jnp.int32),
            pltpu.VMEM((nch, 128), jnp.int32),
            pltpu.VMEM((128,), jnp.int32),
            pltpu.VMEM((128,), jnp.int32),
            pltpu.VMEM((128,), jnp.int32),
            pltpu.VMEM((128,), jnp.int32),
            pltpu.VMEM((128, w), jnp.float32),
            pltpu.VMEM((128, w), jnp.float32),
            pltpu.VMEM((128, w), jnp.float32),
            pltpu.VMEM((128, w), jnp.float32),
            pltpu.VMEM((sro, w), jnp.float32),
            pltpu.SemaphoreType.DMA,
            pltpu.SemaphoreType.DMA,
            pltpu.SemaphoreType.DMA,
            pltpu.SemaphoreType.DMA,
            pltpu.SemaphoreType.DMA,
            pltpu.SemaphoreType.DMA,
            pltpu.SemaphoreType.DMA,
            pltpu.SemaphoreType.DMA,
        ],
        compiler_params=pltpu.CompilerParams(use_tc_tiling_on_sc=False),
    )


_deg = _make_deg()
_spmm_l1 = _make_spmm(8, 16, True)
_spmm_l2 = _make_spmm(4, 32, True)
_spmm_l3 = _make_spmm(4, 32, False)
_spmm_l4 = _make_spmm(4, 32, False)


def _rb(i):
    return (i, 0)


def _full(i):
    return (0, 0)


def _zeros(nlanes):
    return jnp.zeros((RB, nlanes), jnp.float32)


def _tc1_body(d_ref, x_ref, w1_ref, out_ref):
    deg = d_ref[0] + d_ref[1] + 1.0
    dinv = lax.rsqrt(deg)
    g1 = jnp.dot(x_ref[...], w1_ref[...]) * dinv
    out_ref[...] = jnp.concatenate([g1, dinv, _zeros(111)], axis=1)


def _tc1(dstk, x, w1):
    return pl.pallas_call(
        _tc1_body,
        grid=(NRB,),
        in_specs=[
            pl.BlockSpec((NSC, RB, 1), lambda i: (0, i, 0)),
            pl.BlockSpec((RB, 64), _rb),
            pl.BlockSpec((64, 16), _full),
        ],
        out_specs=pl.BlockSpec((RB, 128), _rb),
        out_shape=jax.ShapeDtypeStruct((N, 128), jnp.float32),
    )(dstk, x, w1)


def _tc2_body(s_ref, g_ref, b_ref, w_ref, out_ref):
    dinv = g_ref[:, 16:17]
    s = s_ref[:, :16] + s_ref[:, 16:32] - g_ref[:, :16]
    h = jnp.maximum(s * dinv + b_ref[...], 0.0)
    g2 = jnp.dot(h, w_ref[...]) * dinv
    out_ref[...] = jnp.concatenate([g2, dinv, _zeros(95)], axis=1)


def _tc2(s1, g1pk, b1, w2):
    return pl.pallas_call(
        _tc2_body,
        grid=(NRB,),
        in_specs=[
            pl.BlockSpec((RB, 128), _rb),
            pl.BlockSpec((RB, 128), _rb),
            pl.BlockSpec((1, 16), _full),
            pl.BlockSpec((16, 32), _full),
        ],
        out_specs=pl.BlockSpec((RB, 128), _rb),
        out_shape=jax.ShapeDtypeStruct((N, 128), jnp.float32),
    )(s1, g1pk, b1, w2)


def _tc3a_body(s_ref, g_ref, b_ref, y_ref, p_ref):
    dinv = g_ref[:, 32:33]
    y = (s_ref[:, :32] + s_ref[:, 32:64] - g_ref[:, :32]) * dinv + b_ref[...]
    y_ref[...] = jnp.concatenate([y, dinv, _zeros(95)], axis=1)
    p_ref[0, 0:1, :] = jnp.sum(y, axis=0, keepdims=True)
    p_ref[0, 1:2, :] = jnp.sum(y * y, axis=0, keepdims=True)


def _tc3a(s2, g2pk, b2):
    return pl.pallas_call(
        _tc3a_body,
        grid=(NRB,),
        in_specs=[
            pl.BlockSpec((RB, 128), _rb),
            pl.BlockSpec((RB, 128), _rb),
            pl.BlockSpec((1, 32), _full),
        ],
        out_specs=[
            pl.BlockSpec((RB, 128), _rb),
            pl.BlockSpec((1, 2, 32), lambda i: (i, 0, 0)),
        ],
        out_shape=[
            jax.ShapeDtypeStruct((N, 128), jnp.float32),
            jax.ShapeDtypeStruct((NRB, 2, 32), jnp.float32),
        ],
    )(s2, g2pk, b2)


def _tc3b_body(y_ref, p_ref, gm_ref, bt_ref, w_ref, out_ref):
    dinv = y_ref[:, 32:33]
    m = jnp.sum(p_ref[:, 0, :], axis=0, keepdims=True) / N
    v = jnp.sum(p_ref[:, 1, :], axis=0, keepdims=True) / N - m * m
    scale = gm_ref[...] * lax.rsqrt(v + BN_EPS)
    h = jnp.maximum((y_ref[:, :32] - m) * scale + bt_ref[...], 0.0)
    g3 = jnp.dot(h, w_ref[...]) * dinv
    out_ref[...] = jnp.concatenate([g3, dinv, _zeros(63)], axis=1)


def _tc3b(y2, p2, gm, bt, w3):
    return pl.pallas_call(
        _tc3b_body,
        grid=(NRB,),
        in_specs=[
            pl.BlockSpec((RB, 128), _rb),
            pl.BlockSpec((NRB, 2, 32), lambda i: (0, 0, 0)),
            pl.BlockSpec((1, 32), _full),
            pl.BlockSpec((1, 32), _full),
            pl.BlockSpec((32, 64), _full),
        ],
        out_specs=pl.BlockSpec((RB, 128), _rb),
        out_shape=jax.ShapeDtypeStruct((N, 128), jnp.float32),
    )(y2, p2, gm, bt, w3)


def _tc4_body(s_ref, g_ref, b_ref, w_ref, out_ref):
    dinv = g_ref[:, 64:65]
    h = jnp.maximum(s_ref[:, :64] * dinv + b_ref[...], 0.0)
    out_ref[...] = jnp.dot(h, w_ref[...]) * dinv


def _tc4(s3, g3pk, b3, w4):
    return pl.pallas_call(
        _tc4_body,
        grid=(NRB,),
        in_specs=[
            pl.BlockSpec((RB, 128), _rb),
            pl.BlockSpec((RB, 128), _rb),
            pl.BlockSpec((1, 64), _full),
            pl.BlockSpec((64, 128), _full),
        ],
        out_specs=pl.BlockSpec((RB, 128), _rb),
        out_shape=jax.ShapeDtypeStruct((N, 128), jnp.float32),
    )(s3, g3pk, b3, w4)


def _tc5a_body(s_ref, g_ref, b_ref, y_ref, p_ref):
    dinv = g_ref[:, 64:65]
    y = s_ref[...] * dinv + b_ref[...]
    y_ref[...] = y
    p_ref[0, 0:1, :] = jnp.sum(y, axis=0, keepdims=True)
    p_ref[0, 1:2, :] = jnp.sum(y * y, axis=0, keepdims=True)


def _tc5a(s4, g3pk, b4):
    return pl.pallas_call(
        _tc5a_body,
        grid=(NRB,),
        in_specs=[
            pl.BlockSpec((RB, 128), _rb),
            pl.BlockSpec((RB, 128), _rb),
            pl.BlockSpec((1, 128), _full),
        ],
        out_specs=[
            pl.BlockSpec((RB, 128), _rb),
            pl.BlockSpec((1, 2, 128), lambda i: (i, 0, 0)),
        ],
        out_shape=[
            jax.ShapeDtypeStruct((N, 128), jnp.float32),
            jax.ShapeDtypeStruct((NRB, 2, 128), jnp.float32),
        ],
    )(s4, g3pk, b4)


def _tc5b_body(y_ref, p_ref, gm_ref, bt_ref, wp_ref, bp_ref, out_ref):
    i = pl.program_id(0)
    m = jnp.sum(p_ref[:, 0, :], axis=0, keepdims=True) / N
    v = jnp.sum(p_ref[:, 1, :], axis=0, keepdims=True) / N - m * m
    scale = gm_ref[...] * lax.rsqrt(v + BN_EPS)
    h = jnp.maximum((y_ref[...] - m) * scale + bt_ref[...], 0.0)

    @pl.when(i == 0)
    def _():
        out_ref[...] = jnp.broadcast_to(bp_ref[...], (196, 128))

    out_ref[...] += jnp.dot(wp_ref[...], h)


def _tc5b(y4, p4, gm, bt, wp, bp):
    return pl.pallas_call(
        _tc5b_body,
        grid=(NRB,),
        in_specs=[
            pl.BlockSpec((RB, 128), _rb),
            pl.BlockSpec((NRB, 2, 128), lambda i: (0, 0, 0)),
            pl.BlockSpec((1, 128), _full),
            pl.BlockSpec((1, 128), _full),
            pl.BlockSpec((196, RB), lambda i: (0, i)),
            pl.BlockSpec((196, 1), _full),
        ],
        out_specs=pl.BlockSpec((196, 128), _full),
        out_shape=jax.ShapeDtypeStruct((196, 128), jnp.float32),
    )(y4, p4, gm, bt, wp, bp)


def _tc6_body(vit_ref, p_ref, wl_ref, bl_ref, out_ref):
    wa = wl_ref[:, :768]
    wb = wl_ref[:, 768:]
    dn = (((1,), (1,)), ((), ()))
    out_ref[...] = (lax.dot_general(vit_ref[...], wa, dn)
                    + lax.dot_general(p_ref[...], wb, dn)
                    + bl_ref[...])


def _tc6(vit, p, wl, bl):
    return pl.pallas_call(
        _tc6_body,
        grid=(1,),
        in_specs=[
            pl.BlockSpec((196, 768), _full),
            pl.BlockSpec((196, 128), _full),
            pl.BlockSpec((768, 896), _full),
            pl.BlockSpec((1, 768), _full),
        ],
        out_specs=pl.BlockSpec((196, 768), _full),
        out_shape=jax.ShapeDtypeStruct((196, 768), jnp.float32),
    )(vit, p, wl, bl)


def kernel(x, edge_index, vit_output, W1, b1, W2, b2, W3, b3, W4, b4,
           g1, be1, g2, be2, Wp, bp, Wl, bl):
    src2d = edge_index[0].reshape(EB, 128)
    dst2d = edge_index[1].reshape(EB, 128)

    degp = _deg(dst2d)
    dstk = degp.reshape(NSC, N, 1)
    g1pk = _tc1(dstk, x, W1)

    s1 = _spmm_l1(g1pk.reshape(8 * N, 16), src2d, dst2d)
    g2pk = _tc2(s1, g1pk, b1.reshape(1, 16), W2)

    s2 = _spmm_l2(g2pk.reshape(4 * N, 32), src2d, dst2d)
    y2pk, p2 = _tc3a(s2, g2pk, b2.reshape(1, 32))
    g3pk = _tc3b(y2pk, p2, g1.reshape(1, 32), be1.reshape(1, 32), W3)

    s3 = _spmm_l3(g3pk.reshape(4 * N, 32), src2d, dst2d)
    g4pk = _tc4(s3, g3pk, b3.reshape(1, 64), W4)

    s4 = _spmm_l4(g4pk.reshape(4 * N, 32), src2d, dst2d)
    y4, p4 = _tc5a(s4, g3pk, b4.reshape(1, 128))
    pmat = _tc5b(y4, p4, g2.reshape(1, 128), be2.reshape(1, 128),
                 Wp, bp.reshape(196, 1))

    z = _tc6(vit_output.reshape(196, 768), pmat, Wl, bl.reshape(1, 768))
    return z.reshape(1, 196, 768)


# speedup vs baseline: 28.2125x; 1.1645x over previous
import jax
import jax.numpy as jnp
from jax import lax
from jax.experimental import pallas as pl
from jax.experimental.pallas import tpu as pltpu
from jax.experimental.pallas import tpu_sc as plsc

N = 50176
E = 802816
EB = E // 128
NSC = 2
NTILE = 16
RPT = N // NTILE
NB = 4
BN_EPS = 1e-5
RB = 512
NRB = N // RB

_MESH = plsc.VectorSubcoreMesh(core_axis_name="c", subcore_axis_name="s",
                               num_cores=NSC, num_subcores=NTILE)


def _make_deg():
    def body(dst_hbm, out_hbm, acc, dstv, ones, zbuf, dstage):
        cid = lax.axis_index("c")
        sid = lax.axis_index("s")
        for i in range(8):
            ones[pl.ds(16 * i, 16)] = jnp.full((16,), 1.0, jnp.float32)
        for i in range(14):
            zbuf[pl.ds(16 * i, 16)] = jnp.zeros((16,), jnp.float32)
        r0 = sid * RPT
        for k in range(RPT // 224):
            pltpu.sync_copy(zbuf, acc.at[pl.ds(r0 + k * 224, 224)])
        plsc.subcore_barrier()
        bpt = EB // NSC // NTILE
        nch_t = bpt // 28
        tb = cid * (EB // NSC) + sid * bpt

        def chunk(ci, carry):
            cb = tb + ci * 28
            pltpu.sync_copy(dst_hbm.at[pl.ds(cb, 28)], dstv)

            def bat(j, c2):
                pltpu.sync_copy(ones, acc.at[dstv.at[j]], add=True)
                return c2
            return lax.fori_loop(0, 28, bat, carry)
        lax.fori_loop(0, nch_t, chunk, 0)
        plsc.subcore_barrier()
        pltpu.sync_copy(acc.at[pl.ds(r0, RPT)], dstage)
        pltpu.sync_copy(dstage, out_hbm.at[pl.ds(cid * N + r0, RPT)])

    return pl.kernel(
        body,
        out_type=jax.ShapeDtypeStruct((NSC * N,), jnp.float32),
        mesh=_MESH,
        scratch_types=[
            pltpu.VMEM_SHARED((N,), jnp.float32),
            pltpu.VMEM((28, 128), jnp.int32),
            pltpu.VMEM((128,), jnp.float32),
            pltpu.VMEM((224,), jnp.float32),
            pltpu.VMEM((RPT,), jnp.float32),
        ],
        compiler_params=pltpu.CompilerParams(use_tc_tiling_on_sc=False),
    )


def _make_spmm(sp, w, edge_split, nblk=1):
    shift = sp.bit_length() - 1
    blocks_per_sc = 1 if edge_split else nblk // NSC
    nch = 28
    sro = 112

    def body(gf_hbm, src_hbm, dst_hbm, out_hbm, acc, srcv, dstv,
             idx0, idx1, idx2, idx3, rows0, rows1, rows2, rows3, stage,
             gsem0, gsem1, gsem2, gsem3, ssem0, ssem1, ssem2, ssem3):
        cid = lax.axis_index("c")
        sid = lax.axis_index("s")
        r0 = sid * RPT
        rows = (rows0, rows1, rows2, rows3)
        idxb = (idx0, idx1, idx2, idx3)
        gsem = (gsem0, gsem1, gsem2, gsem3)
        ssem = (ssem0, ssem1, ssem2, ssem3)
        for t in range(blocks_per_sc):
            if edge_split:
                part = cid
                gpart = 0
                nch_t = (EB // NSC // NTILE) // nch
                tb = cid * (EB // NSC) + sid * (EB // NSC // NTILE)
            else:
                part = cid * blocks_per_sc + t
                gpart = part
                nch_t = (EB // NTILE) // nch
                tb = sid * (EB // NTILE)
            lanes = lax.iota(jnp.int32, 16)

            def initk(k, carry):
                base = r0 + k * 112
                for i in range(7):
                    idx0[pl.ds(16 * i, 16)] = (
                        ((base + 16 * i + lanes) << shift) + gpart)
                pltpu.async_copy(gf_hbm.at[idx0.at[pl.ds(0, 112)]],
                                 stage.at[pl.ds(0, 112)], gsem0).wait()
                pltpu.sync_copy(stage.at[pl.ds(0, 112)],
                                acc.at[pl.ds(base, 112)])
                return carry
            lax.fori_loop(0, RPT // 112, initk, 0)
            plsc.subcore_barrier()

            def issue_gather(j, b):
                for i in range(8):
                    v = srcv[j, pl.ds(16 * i, 16)] << shift
                    if not edge_split:
                        v = v + gpart
                    idxb[b][pl.ds(16 * i, 16)] = v
                return pltpu.async_copy(gf_hbm.at[idxb[b]], rows[b], gsem[b])

            def chunk(ci, carry):
                cb = tb + ci * nch
                c1 = pltpu.async_copy(src_hbm.at[pl.ds(cb, nch)], srcv,
                                      gsem0)
                c2 = pltpu.async_copy(dst_hbm.at[pl.ds(cb, nch)], dstv,
                                      gsem1)
                c1.wait()
                c2.wait()
                gh = [None] * NB
                sh = [None] * NB
                for k in range(min(NB - 1, nch)):
                    gh[k] = issue_gather(k, k)
                for j in range(nch):
                    b = j % NB
                    gh[b].wait()
                    sh[b] = pltpu.async_copy(rows[b], acc.at[dstv.at[j]],
                                             ssem[b], add=True)
                    if j >= 1:
                        sh[(j - 1) % NB].wait()
                    if j + NB - 1 < nch:
                        b2 = (j + NB - 1) % NB
                        gh[b2] = issue_gather(j + NB - 1, b2)
                sh[(nch - 1) % NB].wait()
                return carry
            lax.fori_loop(0, nch_t, chunk, 0)
            plsc.subcore_barrier()

            def outk(k, carry):
                pltpu.sync_copy(acc.at[pl.ds(r0 + k * sro, sro)], stage)
                pltpu.sync_copy(
                    stage,
                    out_hbm.at[pl.ds(r0 + k * sro, sro), pl.ds(part * w, w)])
                return carry
            lax.fori_loop(0, RPT // sro, outk, 0)

    return pl.kernel(
        body,
        out_type=jax.ShapeDtypeStruct((N, 128), jnp.float32),
        mesh=_MESH,
        scratch_types=[
            pltpu.VMEM_SHARED((N, w), jnp.float32),
            pltpu.VMEM((nch, 128), jnp.int32),
            pltpu.VMEM((nch, 128), jnp.int32),
            pltpu.VMEM((128,), jnp.int32),
            pltpu.VMEM((128,), jnp.int32),
            pltpu.VMEM((128,), jnp.int32),
            pltpu.VMEM((128,), jnp.int32),
            pltpu.VMEM((128, w), jnp.float32),
            pltpu.VMEM((128, w), jnp.float32),
            pltpu.VMEM((128, w), jnp.float32),
            pltpu.VMEM((128, w), jnp.float32),
            pltpu.VMEM((sro, w), jnp.float32),
            pltpu.SemaphoreType.DMA,
            pltpu.SemaphoreType.DMA,
            pltpu.SemaphoreType.DMA,
            pltpu.SemaphoreType.DMA,
            pltpu.SemaphoreType.DMA,
            pltpu.SemaphoreType.DMA,
            pltpu.SemaphoreType.DMA,
            pltpu.SemaphoreType.DMA,
        ],
        compiler_params=pltpu.CompilerParams(use_tc_tiling_on_sc=False),
    )


_deg = _make_deg()
_spmm_l1 = _make_spmm(8, 16, True)
_spmm_l2 = _make_spmm(4, 32, True)
_spmm_l3 = _make_spmm(4, 32, False, nblk=2)
_spmm_l4 = _make_spmm(4, 32, False, nblk=4)


def _rb(i):
    return (i, 0)


def _full(i):
    return (0, 0)


def _zeros(nlanes):
    return jnp.zeros((RB, nlanes), jnp.float32)


def _tc1_body(d_ref, x_ref, w1_ref, out_ref):
    deg = d_ref[0] + d_ref[1] + 1.0
    dinv = lax.rsqrt(deg)
    g1 = jnp.dot(x_ref[...], w1_ref[...]) * dinv
    out_ref[...] = jnp.concatenate([g1, dinv, _zeros(111)], axis=1)


def _tc1(dstk, x, w1):
    return pl.pallas_call(
        _tc1_body,
        grid=(NRB,),
        in_specs=[
            pl.BlockSpec((NSC, RB, 1), lambda i: (0, i, 0)),
            pl.BlockSpec((RB, 64), _rb),
            pl.BlockSpec((64, 16), _full),
        ],
        out_specs=pl.BlockSpec((RB, 128), _rb),
        out_shape=jax.ShapeDtypeStruct((N, 128), jnp.float32),
    )(dstk, x, w1)


def _tc2_body(s_ref, g_ref, b_ref, w_ref, out_ref):
    dinv = g_ref[:, 16:17]
    s = s_ref[:, :16] + s_ref[:, 16:32] - g_ref[:, :16]
    h = jnp.maximum(s * dinv + b_ref[...], 0.0)
    g2 = jnp.dot(h, w_ref[...]) * dinv
    out_ref[...] = jnp.concatenate([g2, dinv, _zeros(95)], axis=1)


def _tc2(s1, g1pk, b1, w2):
    return pl.pallas_call(
        _tc2_body,
        grid=(NRB,),
        in_specs=[
            pl.BlockSpec((RB, 128), _rb),
            pl.BlockSpec((RB, 128), _rb),
            pl.BlockSpec((1, 16), _full),
            pl.BlockSpec((16, 32), _full),
        ],
        out_specs=pl.BlockSpec((RB, 128), _rb),
        out_shape=jax.ShapeDtypeStruct((N, 128), jnp.float32),
    )(s1, g1pk, b1, w2)


def _tc3a_body(s_ref, g_ref, b_ref, y_ref, p_ref):
    dinv = g_ref[:, 32:33]
    y = (s_ref[:, :32] + s_ref[:, 32:64] - g_ref[:, :32]) * dinv + b_ref[...]
    y_ref[...] = jnp.concatenate([y, dinv, _zeros(95)], axis=1)
    p_ref[0, 0:1, :] = jnp.sum(y, axis=0, keepdims=True)
    p_ref[0, 1:2, :] = jnp.sum(y * y, axis=0, keepdims=True)


def _tc3a(s2, g2pk, b2):
    return pl.pallas_call(
        _tc3a_body,
        grid=(NRB,),
        in_specs=[
            pl.BlockSpec((RB, 128), _rb),
            pl.BlockSpec((RB, 128), _rb),
            pl.BlockSpec((1, 32), _full),
        ],
        out_specs=[
            pl.BlockSpec((RB, 128), _rb),
            pl.BlockSpec((1, 2, 32), lambda i: (i, 0, 0)),
        ],
        out_shape=[
            jax.ShapeDtypeStruct((N, 128), jnp.float32),
            jax.ShapeDtypeStruct((NRB, 2, 32), jnp.float32),
        ],
    )(s2, g2pk, b2)


def _tc3b_body(y_ref, p_ref, gm_ref, bt_ref, w_ref, out_ref):
    dinv = y_ref[:, 32:33]
    m = jnp.sum(p_ref[:, 0, :], axis=0, keepdims=True) / N
    v = jnp.sum(p_ref[:, 1, :], axis=0, keepdims=True) / N - m * m
    scale = gm_ref[...] * lax.rsqrt(v + BN_EPS)
    h = jnp.maximum((y_ref[:, :32] - m) * scale + bt_ref[...], 0.0)
    g3 = jnp.dot(h, w_ref[...]) * dinv
    out_ref[...] = jnp.concatenate([g3, dinv, _zeros(63)], axis=1)


def _tc3b(y2, p2, gm, bt, w3):
    return pl.pallas_call(
        _tc3b_body,
        grid=(NRB,),
        in_specs=[
            pl.BlockSpec((RB, 128), _rb),
            pl.BlockSpec((NRB, 2, 32), lambda i: (0, 0, 0)),
            pl.BlockSpec((1, 32), _full),
            pl.BlockSpec((1, 32), _full),
            pl.BlockSpec((32, 64), _full),
        ],
        out_specs=pl.BlockSpec((RB, 128), _rb),
        out_shape=jax.ShapeDtypeStruct((N, 128), jnp.float32),
    )(y2, p2, gm, bt, w3)


def _tc4_body(s_ref, g_ref, b_ref, w_ref, out_ref):
    dinv = g_ref[:, 64:65]
    h = jnp.maximum(s_ref[:, :64] * dinv + b_ref[...], 0.0)
    out_ref[...] = jnp.dot(h, w_ref[...]) * dinv


def _tc4(s3, g3pk, b3, w4):
    return pl.pallas_call(
        _tc4_body,
        grid=(NRB,),
        in_specs=[
            pl.BlockSpec((RB, 128), _rb),
            pl.BlockSpec((RB, 128), _rb),
            pl.BlockSpec((1, 64), _full),
            pl.BlockSpec((64, 128), _full),
        ],
        out_specs=pl.BlockSpec((RB, 128), _rb),
        out_shape=jax.ShapeDtypeStruct((N, 128), jnp.float32),
    )(s3, g3pk, b3, w4)


def _tc5a_body(s_ref, g_ref, b_ref, y_ref, p_ref):
    dinv = g_ref[:, 64:65]
    y = s_ref[...] * dinv + b_ref[...]
    y_ref[...] = y
    p_ref[0, 0:1, :] = jnp.sum(y, axis=0, keepdims=True)
    p_ref[0, 1:2, :] = jnp.sum(y * y, axis=0, keepdims=True)


def _tc5a(s4, g3pk, b4):
    return pl.pallas_call(
        _tc5a_body,
        grid=(NRB,),
        in_specs=[
            pl.BlockSpec((RB, 128), _rb),
            pl.BlockSpec((RB, 128), _rb),
            pl.BlockSpec((1, 128), _full),
        ],
        out_specs=[
            pl.BlockSpec((RB, 128), _rb),
            pl.BlockSpec((1, 2, 128), lambda i: (i, 0, 0)),
        ],
        out_shape=[
            jax.ShapeDtypeStruct((N, 128), jnp.float32),
            jax.ShapeDtypeStruct((NRB, 2, 128), jnp.float32),
        ],
    )(s4, g3pk, b4)


def _tc5b_body(y_ref, p_ref, gm_ref, bt_ref, wp_ref, bp_ref, out_ref):
    i = pl.program_id(0)
    m = jnp.sum(p_ref[:, 0, :], axis=0, keepdims=True) / N
    v = jnp.sum(p_ref[:, 1, :], axis=0, keepdims=True) / N - m * m
    scale = gm_ref[...] * lax.rsqrt(v + BN_EPS)
    h = jnp.maximum((y_ref[...] - m) * scale + bt_ref[...], 0.0)

    @pl.when(i == 0)
    def _():
        out_ref[...] = jnp.broadcast_to(bp_ref[...], (196, 128))

    out_ref[...] += jnp.dot(wp_ref[...], h)


def _tc5b(y4, p4, gm, bt, wp, bp):
    return pl.pallas_call(
        _tc5b_body,
        grid=(NRB,),
        in_specs=[
            pl.BlockSpec((RB, 128), _rb),
            pl.BlockSpec((NRB, 2, 128), lambda i: (0, 0, 0)),
            pl.BlockSpec((1, 128), _full),
            pl.BlockSpec((1, 128), _full),
            pl.BlockSpec((196, RB), lambda i: (0, i)),
            pl.BlockSpec((196, 1), _full),
        ],
        out_specs=pl.BlockSpec((196, 128), _full),
        out_shape=jax.ShapeDtypeStruct((196, 128), jnp.float32),
    )(y4, p4, gm, bt, wp, bp)


def _tc6_body(vit_ref, p_ref, wl_ref, bl_ref, out_ref):
    wa = wl_ref[:, :768]
    wb = wl_ref[:, 768:]
    dn = (((1,), (1,)), ((), ()))
    out_ref[...] = (lax.dot_general(vit_ref[...], wa, dn)
                    + lax.dot_general(p_ref[...], wb, dn)
                    + bl_ref[...])


def _tc6(vit, p, wl, bl):
    return pl.pallas_call(
        _tc6_body,
        grid=(1,),
        in_specs=[
            pl.BlockSpec((196, 768), _full),
            pl.BlockSpec((196, 128), _full),
            pl.BlockSpec((768, 896), _full),
            pl.BlockSpec((1, 768), _full),
        ],
        out_specs=pl.BlockSpec((196, 768), _full),
        out_shape=jax.ShapeDtypeStruct((196, 768), jnp.float32),
    )(vit, p, wl, bl)


def kernel(x, edge_index, vit_output, W1, b1, W2, b2, W3, b3, W4, b4,
           g1, be1, g2, be2, Wp, bp, Wl, bl):
    src2d = edge_index[0].reshape(EB, 128)
    dst2d = edge_index[1].reshape(EB, 128)

    degp = _deg(dst2d)
    dstk = degp.reshape(NSC, N, 1)
    g1pk = _tc1(dstk, x, W1)

    s1 = _spmm_l1(g1pk.reshape(8 * N, 16), src2d, dst2d)
    g2pk = _tc2(s1, g1pk, b1.reshape(1, 16), W2)

    s2 = _spmm_l2(g2pk.reshape(4 * N, 32), src2d, dst2d)
    y2pk, p2 = _tc3a(s2, g2pk, b2.reshape(1, 32))
    g3pk = _tc3b(y2pk, p2, g1.reshape(1, 32), be1.reshape(1, 32), W3)

    s3 = _spmm_l3(g3pk.reshape(4 * N, 32), src2d, dst2d)
    g4pk = _tc4(s3, g3pk, b3.reshape(1, 64), W4)

    s4 = _spmm_l4(g4pk.reshape(4 * N, 32), src2d, dst2d)
    y4, p4 = _tc5a(s4, g3pk, b4.reshape(1, 128))
    pmat = _tc5b(y4, p4, g2.reshape(1, 128), be2.reshape(1, 128),
                 Wp, bp.reshape(196, 1))

    z = _tc6(vit_output.reshape(196, 768), pmat, Wl, bl.reshape(1, 768))
    return z.reshape(1, 196, 768)
